# baseline (speedup 1.0000x reference)
import jax
import jax.numpy as jnp
from jax import lax
from jax.experimental import pallas as pl
from jax.experimental.pallas import tpu as pltpu
from jax.experimental.pallas import tpu_sc as plsc

NUM_ITEMS = 100000
D = 128
B = 4096

_SC_INFO = plsc.get_sparse_core_info()
_NC = _SC_INFO.num_cores
_NS = _SC_INFO.num_subcores
_NW = _NC * _NS
_B_PER_W = B // _NW


def _sc_gather_body(table_hbm, idx_hbm, out_hbm, idx_v, rows_v, sem):
    wid = lax.axis_index("s") * _NC + lax.axis_index("c")
    base = wid * _B_PER_W
    pltpu.sync_copy(idx_hbm.at[pl.ds(base, _B_PER_W)], idx_v)
    pltpu.async_copy(table_hbm.at[idx_v], rows_v, sem).wait()
    pltpu.sync_copy(rows_v, out_hbm.at[pl.ds(base, _B_PER_W)])


_sc_gather = pl.kernel(
    _sc_gather_body,
    out_type=jax.ShapeDtypeStruct((B, D), jnp.float32),
    mesh=plsc.VectorSubcoreMesh(core_axis_name="c", subcore_axis_name="s"),
    scratch_types=[
        pltpu.VMEM((_B_PER_W,), jnp.int32),
        pltpu.VMEM((_B_PER_W, D), jnp.float32),
        pltpu.SemaphoreType.DMA,
    ],
)

_TI = 512


def _mm_body(u_ref, it_ref, o_ref):
    s = lax.dot_general(
        u_ref[...], it_ref[...],
        dimension_numbers=(((1,), (1,)), ((), ())),
        preferred_element_type=jnp.float32,
    )
    o_ref[...] = 1.0 / (1.0 + jnp.exp(-s))


@jax.jit
def kernel(users, user_table, item_table):
    users_emb = _sc_gather(user_table, users)
    return pl.pallas_call(
        _mm_body,
        grid=(pl.cdiv(NUM_ITEMS, _TI),),
        in_specs=[
            pl.BlockSpec((B, D), lambda i: (0, 0)),
            pl.BlockSpec((_TI, D), lambda i: (i, 0)),
        ],
        out_specs=pl.BlockSpec((B, _TI), lambda i: (0, i)),
        out_shape=jax.ShapeDtypeStruct((B, NUM_ITEMS), jnp.float32),
    )(users_emb, item_table)
